# baseline (speedup 1.0000x reference)
import functools

import jax
import jax.numpy as jnp
from jax import lax
from jax.experimental import pallas as pl
from jax.experimental.pallas import tpu as pltpu
from jax.experimental.pallas import tpu_sc as plsc

B = 16384
D = 128
W = D // 2
N_POS = 10
N_NEG = 50
J = N_POS + N_NEG
NC = 2
NS = 16
NW = NC * NS
EPW = B // NW
PPW = EPW * J
ROWS = 128
NCH = PPW // ROWS
BLK = 8
NBLK = NCH // BLK
NBUF = 8
ERB = 32


def _sc_dots(in_tab_w, out_tab_w, in_lbl, out_lbl):
    mesh = plsc.VectorSubcoreMesh(core_axis_name="c", subcore_axis_name="s")

    @functools.partial(
        pl.kernel,
        out_type=jax.ShapeDtypeStruct((B * J,), jnp.float32),
        mesh=mesh,
        compiler_params=pltpu.CompilerParams(needs_layout_passes=False,
                                             use_tc_tiling_on_sc=False),
        scratch_types=[
            pltpu.VMEM((4, 128), jnp.int32),
            pltpu.VMEM((2 * ERB, W), jnp.float32),
            pltpu.VMEM((2 * BLK, 128), jnp.int32),
            pltpu.VMEM((NBUF * ROWS, W), jnp.float32),
            pltpu.VMEM((2, BLK * ROWS), jnp.float32),
            pltpu.SemaphoreType.DMA,
            pltpu.SemaphoreType.DMA,
            pltpu.SemaphoreType.DMA,
            pltpu.SemaphoreType.DMA,
            pltpu.SemaphoreType.DMA,
            pltpu.SemaphoreType.DMA,
            pltpu.SemaphoreType.DMA,
            pltpu.SemaphoreType.DMA,
            pltpu.SemaphoreType.DMA,
            pltpu.SemaphoreType.DMA,
            pltpu.SemaphoreType.DMA,
            pltpu.SemaphoreType.DMA,
            pltpu.SemaphoreType.DMA,
            pltpu.SemaphoreType.DMA,
        ],
    )
    def k(in_tab, out_tab, in_lbl_h, out_lbl_h, dots_h,
          in_lbl_v, in_rows_v, idx_v, rows_v, dots_v,
          sq0, sq1, si0, si1, sr0, sr1, sr2, sr3, sr4, sr5, sr6, sr7, sd0, sd1):
        sem_q = [sq0, sq1]
        sem_i = [si0, si1]
        sem_r = [sr0, sr1, sr2, sr3, sr4, sr5, sr6, sr7]
        sem_d = [sd0, sd1]
        wid = lax.axis_index("s") * NC + lax.axis_index("c")
        ibase = wid * NCH
        lane = lax.iota(jnp.int32, 16)

        def issue_in_rows(blk_idx, dst_half, sem):
            est = blk_idx * (BLK * ROWS) // J
            for kk in range(ERB // 16):
                pos = jnp.minimum(est + kk * 16 + lane, EPW - 1)
                lbl = plsc.load_gather(in_lbl_v, [pos // 128, pos % 128])
                pltpu.async_copy(
                    in_tab.at[lbl],
                    in_rows_v.at[pl.ds(dst_half * ERB + kk * 16, 16)], sem)

        def wait_in_rows(dst_half, sem):
            for kk in range(ERB // 16):
                pltpu.make_async_copy(
                    in_tab.at[in_lbl_v[0, pl.ds(0, 16)]],
                    in_rows_v.at[pl.ds(dst_half * ERB + kk * 16, 16)],
                    sem).wait()

        for kk in range(4):
            pltpu.sync_copy(in_lbl_h.at[wid * 4 + kk], in_lbl_v.at[kk])
        issue_in_rows(jnp.int32(0), 0, sem_q[0])
        pltpu.sync_copy(out_lbl_h.at[pl.ds(ibase, BLK)],
                        idx_v.at[pl.ds(0, BLK)])
        for kk in range(NBUF - 1):
            pltpu.async_copy(out_tab.at[idx_v.at[kk]],
                             rows_v.at[pl.ds(kk * ROWS, ROWS)], sem_r[kk])

        def compute_chunk(t, est, pr, pb, u):
            ridx = [lane + (pr * ROWS + g * 16) for g in range(ROWS // 16)]
            brow = []
            for g in range(ROWS // 16):
                q = t * ROWS + g * 16 + lane
                brow.append(pb * ERB + (q // J - est))

            def d_body(d, accs):
                col = (jnp.full((16,), d, jnp.int32) + lane) & (W - 1)
                out = []
                for g in range(ROWS // 16):
                    bw = plsc.load_gather(in_rows_v, [brow[g], col])
                    vw = plsc.load_gather(rows_v, [ridx[g], col])
                    b_lo, b_hi = plsc.unpack(
                        plsc.bitcast(bw, jnp.bfloat16),
                        format=plsc.PackFormat.INTERLEAVED)
                    v_lo, v_hi = plsc.unpack(
                        plsc.bitcast(vw, jnp.bfloat16),
                        format=plsc.PackFormat.INTERLEAVED)
                    out.append(accs[g] + (v_lo * b_lo + v_hi * b_hi))
                return tuple(out)

            zero = jnp.zeros((16,), jnp.float32)
            accs = lax.fori_loop(0, W, d_body, (zero,) * (ROWS // 16),
                                 unroll=2)
            for g in range(ROWS // 16):
                dots_v[pb, pl.ds(u * ROWS + g * 16, 16)] = accs[g]

        def outer(ot, _):
            for pb in range(2):
                bi = ot * 2 + pb
                nxt = jnp.minimum(bi + 1, NBLK - 1)
                pltpu.async_copy(
                    out_lbl_h.at[pl.ds(ibase + nxt * BLK, BLK)],
                    idx_v.at[pl.ds((1 - pb) * BLK, BLK)], sem_i[1 - pb])
                wait_in_rows(pb, sem_q[pb])
                issue_in_rows(nxt, 1 - pb, sem_q[1 - pb])
                est = bi * (BLK * ROWS) // J

                @pl.when(bi >= 2)
                def _():
                    pltpu.make_async_copy(
                        dots_v.at[pb],
                        dots_h.at[pl.ds(0, BLK * ROWS)], sem_d[pb]).wait()

                for u in range(BLK):
                    t = bi * BLK + u
                    pr = u % NBUF
                    if u == BLK - NBUF + 1:
                        pltpu.make_async_copy(
                            out_lbl_h.at[pl.ds(ibase, BLK)],
                            idx_v.at[pl.ds((1 - pb) * BLK, BLK)],
                            sem_i[1 - pb]).wait()
                    if u <= BLK - NBUF:
                        nidx = idx_v.at[pb * BLK + u + NBUF - 1]
                    else:
                        nidx = idx_v.at[(1 - pb) * BLK + u + NBUF - 1 - BLK]

                    @pl.when(t + NBUF - 1 < NCH)
                    def _(nidx=nidx, dst=(u + NBUF - 1) % NBUF):
                        pltpu.async_copy(
                            out_tab.at[nidx],
                            rows_v.at[pl.ds(dst * ROWS, ROWS)],
                            sem_r[dst])
                    pltpu.make_async_copy(
                        out_tab.at[idx_v.at[0]],
                        rows_v.at[pl.ds(pr * ROWS, ROWS)],
                        sem_r[pr]).wait()
                    compute_chunk(t, est, pr, pb, u)

                goff = (ibase + bi * BLK) * ROWS
                pltpu.async_copy(dots_v.at[pb],
                                 dots_h.at[pl.ds(goff, BLK * ROWS)],
                                 sem_d[pb])
            return 0

        lax.fori_loop(0, NBLK // 2, outer, 0)

        wait_in_rows(0, sem_q[0])
        for pb in range(2):
            pltpu.make_async_copy(dots_v.at[pb],
                                  dots_h.at[pl.ds(0, BLK * ROWS)],
                                  sem_d[pb]).wait()

    return k(in_tab_w, out_tab_w, in_lbl, out_lbl)


def _tc_loss(dots):
    def body(d_ref, o_ref):
        d = d_ref[...]
        j = lax.broadcasted_iota(jnp.int32, d.shape, 1)
        x = jnp.where(j < N_POS, d, -d)
        ls = jnp.minimum(x, 0.0) - jnp.log1p(jnp.exp(-jnp.abs(x)))
        o_ref[...] = -jnp.sum(ls, axis=1)

    blk = 1024
    return pl.pallas_call(
        body,
        grid=(B // blk,),
        in_specs=[pl.BlockSpec((blk, J), lambda i: (i, 0))],
        out_specs=pl.BlockSpec((blk,), lambda i: (i,)),
        out_shape=jax.ShapeDtypeStruct((B,), jnp.float32),
    )(dots)


def _pack_words(tab):
    bf = tab.astype(jnp.bfloat16).reshape(tab.shape[0], W, 2)
    return jax.lax.bitcast_convert_type(bf, jnp.float32)


@jax.jit
def kernel(input_labels, positive_labels, negative_labels, input_table,
           output_table):
    out_lbl = jnp.concatenate(
        [positive_labels, negative_labels], axis=1).reshape(B * J // 128, 128)
    in_lbl = input_labels.reshape(B // 128, 128)
    dots = _sc_dots(_pack_words(input_table), _pack_words(output_table),
                    in_lbl, out_lbl)
    return _tc_loss(dots.reshape(B, J))


# speedup vs baseline: 1.0100x; 1.0100x over previous
import functools

import jax
import jax.numpy as jnp
from jax import lax
from jax.experimental import pallas as pl
from jax.experimental.pallas import tpu as pltpu
from jax.experimental.pallas import tpu_sc as plsc

B = 16384
D = 128
W = D // 2
N_POS = 10
N_NEG = 50
J = N_POS + N_NEG
NC = 2
NS = 16
NW = NC * NS
EPW = B // NW
PPW = EPW * J
ROWS = 128
NCH = PPW // ROWS
BLK = 8
NBLK = NCH // BLK
NBUF = 4
ERB = 32


def _sc_dots(in_tab_w, out_tab_w, in_lbl, out_lbl):
    mesh = plsc.VectorSubcoreMesh(core_axis_name="c", subcore_axis_name="s")

    @functools.partial(
        pl.kernel,
        out_type=jax.ShapeDtypeStruct((B * J,), jnp.float32),
        mesh=mesh,
        compiler_params=pltpu.CompilerParams(needs_layout_passes=False,
                                             use_tc_tiling_on_sc=False),
        scratch_types=[
            pltpu.VMEM((4, 128), jnp.int32),
            pltpu.VMEM((2 * ERB, W), jnp.float32),
            pltpu.VMEM((2 * BLK, 128), jnp.int32),
            pltpu.VMEM((NBUF * ROWS, W), jnp.float32),
            pltpu.VMEM((2, BLK * ROWS), jnp.float32),
            pltpu.SemaphoreType.DMA,
            pltpu.SemaphoreType.DMA,
            pltpu.SemaphoreType.DMA,
            pltpu.SemaphoreType.DMA,
            pltpu.SemaphoreType.DMA,
            pltpu.SemaphoreType.DMA,
            pltpu.SemaphoreType.DMA,
            pltpu.SemaphoreType.DMA,
            pltpu.SemaphoreType.DMA,
            pltpu.SemaphoreType.DMA,
        ],
    )
    def k(in_tab, out_tab, in_lbl_h, out_lbl_h, dots_h,
          in_lbl_v, in_rows_v, idx_v, rows_v, dots_v,
          sq0, sq1, si0, si1, sr0, sr1, sr2, sr3, sd0, sd1):
        sem_q = [sq0, sq1]
        sem_i = [si0, si1]
        sem_r = [sr0, sr1, sr2, sr3]
        sem_d = [sd0, sd1]
        wid = lax.axis_index("s") * NC + lax.axis_index("c")
        ibase = wid * NCH
        lane = lax.iota(jnp.int32, 16)

        def issue_in_rows(blk_idx, dst_half, sem):
            est = blk_idx * (BLK * ROWS) // J
            for kk in range(ERB // 16):
                pos = jnp.minimum(est + kk * 16 + lane, EPW - 1)
                lbl = plsc.load_gather(in_lbl_v, [pos // 128, pos % 128])
                pltpu.async_copy(
                    in_tab.at[lbl],
                    in_rows_v.at[pl.ds(dst_half * ERB + kk * 16, 16)], sem)

        def wait_in_rows(dst_half, sem):
            for kk in range(ERB // 16):
                pltpu.make_async_copy(
                    in_tab.at[in_lbl_v[0, pl.ds(0, 16)]],
                    in_rows_v.at[pl.ds(dst_half * ERB + kk * 16, 16)],
                    sem).wait()

        for kk in range(4):
            pltpu.sync_copy(in_lbl_h.at[wid * 4 + kk], in_lbl_v.at[kk])
        issue_in_rows(jnp.int32(0), 0, sem_q[0])
        pltpu.sync_copy(out_lbl_h.at[pl.ds(ibase, BLK)],
                        idx_v.at[pl.ds(0, BLK)])
        for kk in range(NBUF - 1):
            pltpu.async_copy(out_tab.at[idx_v.at[kk]],
                             rows_v.at[pl.ds(kk * ROWS, ROWS)], sem_r[kk])

        def compute_chunk(t, est, pr, pb, u):
            ridx = [lane + (pr * ROWS + g * 16) for g in range(ROWS // 16)]
            brow = []
            for g in range(ROWS // 16):
                q = t * ROWS + g * 16 + lane
                brow.append(pb * ERB + (q // J - est))

            def d_body(d, accs):
                col = (jnp.full((16,), d, jnp.int32) + lane) & (W - 1)
                out = []
                for g in range(ROWS // 16):
                    bw = plsc.load_gather(in_rows_v, [brow[g], col])
                    vw = plsc.load_gather(rows_v, [ridx[g], col])
                    b_lo, b_hi = plsc.unpack(
                        plsc.bitcast(bw, jnp.bfloat16),
                        format=plsc.PackFormat.INTERLEAVED)
                    v_lo, v_hi = plsc.unpack(
                        plsc.bitcast(vw, jnp.bfloat16),
                        format=plsc.PackFormat.INTERLEAVED)
                    out.append(accs[g] + (v_lo * b_lo + v_hi * b_hi))
                return tuple(out)

            zero = jnp.zeros((16,), jnp.float32)
            accs = lax.fori_loop(0, W, d_body, (zero,) * (ROWS // 16),
                                 unroll=2)
            for g in range(ROWS // 16):
                dots_v[pb, pl.ds(u * ROWS + g * 16, 16)] = accs[g]

        def outer(ot, _):
            for pb in range(2):
                bi = ot * 2 + pb
                nxt = jnp.minimum(bi + 1, NBLK - 1)
                pltpu.async_copy(
                    out_lbl_h.at[pl.ds(ibase + nxt * BLK, BLK)],
                    idx_v.at[pl.ds((1 - pb) * BLK, BLK)], sem_i[1 - pb])
                wait_in_rows(pb, sem_q[pb])
                issue_in_rows(nxt, 1 - pb, sem_q[1 - pb])
                est = bi * (BLK * ROWS) // J

                @pl.when(bi >= 2)
                def _():
                    pltpu.make_async_copy(
                        dots_v.at[pb],
                        dots_h.at[pl.ds(0, BLK * ROWS)], sem_d[pb]).wait()

                for u in range(BLK):
                    t = bi * BLK + u
                    pr = u % NBUF
                    if u == BLK - NBUF + 1:
                        pltpu.make_async_copy(
                            out_lbl_h.at[pl.ds(ibase, BLK)],
                            idx_v.at[pl.ds((1 - pb) * BLK, BLK)],
                            sem_i[1 - pb]).wait()
                    if u <= BLK - NBUF:
                        nidx = idx_v.at[pb * BLK + u + NBUF - 1]
                    else:
                        nidx = idx_v.at[(1 - pb) * BLK + u + NBUF - 1 - BLK]

                    @pl.when(t + NBUF - 1 < NCH)
                    def _(nidx=nidx, dst=(u + NBUF - 1) % NBUF):
                        pltpu.async_copy(
                            out_tab.at[nidx],
                            rows_v.at[pl.ds(dst * ROWS, ROWS)],
                            sem_r[dst])
                    pltpu.make_async_copy(
                        out_tab.at[idx_v.at[0]],
                        rows_v.at[pl.ds(pr * ROWS, ROWS)],
                        sem_r[pr]).wait()
                    compute_chunk(t, est, pr, pb, u)

                goff = (ibase + bi * BLK) * ROWS
                pltpu.async_copy(dots_v.at[pb],
                                 dots_h.at[pl.ds(goff, BLK * ROWS)],
                                 sem_d[pb])
            return 0

        lax.fori_loop(0, NBLK // 2, outer, 0)

        wait_in_rows(0, sem_q[0])
        for pb in range(2):
            pltpu.make_async_copy(dots_v.at[pb],
                                  dots_h.at[pl.ds(0, BLK * ROWS)],
                                  sem_d[pb]).wait()

    return k(in_tab_w, out_tab_w, in_lbl, out_lbl)


def _tc_loss(dots):
    def body(d_ref, o_ref):
        d = d_ref[...]
        j = lax.broadcasted_iota(jnp.int32, d.shape, 1)
        x = jnp.where(j < N_POS, d, -d)
        ls = jnp.minimum(x, 0.0) - jnp.log1p(jnp.exp(-jnp.abs(x)))
        o_ref[...] = -jnp.sum(ls, axis=1)

    blk = 1024
    return pl.pallas_call(
        body,
        grid=(B // blk,),
        in_specs=[pl.BlockSpec((blk, J), lambda i: (i, 0))],
        out_specs=pl.BlockSpec((blk,), lambda i: (i,)),
        out_shape=jax.ShapeDtypeStruct((B,), jnp.float32),
    )(dots)


def _pack_words(tab):
    bf = tab.astype(jnp.bfloat16).reshape(tab.shape[0], W, 2)
    return jax.lax.bitcast_convert_type(bf, jnp.float32)


@jax.jit
def kernel(input_labels, positive_labels, negative_labels, input_table,
           output_table):
    out_lbl = jnp.concatenate(
        [positive_labels, negative_labels], axis=1).reshape(B * J // 128, 128)
    in_lbl = input_labels.reshape(B // 128, 128)
    dots = _sc_dots(_pack_words(input_table), _pack_words(output_table),
                    in_lbl, out_lbl)
    return _tc_loss(dots.reshape(B, J))
